# baseline (speedup 1.0000x reference)
import jax
import jax.numpy as jnp
from jax.experimental import pallas as pl
from jax.experimental.pallas import tpu as pltpu

_N = 131072
_D = 256
_BLK = 8192
_CHUNK = 128


def _scan_kernel(q_ref, k_ref, v_ref, o_ref, m_s, u_s, w_s):
    i = pl.program_id(0)

    @pl.when(i == 0)
    def _():
        m_s[...] = jnp.full_like(m_s, -jnp.inf)
        u_s[...] = jnp.zeros_like(u_s)
        w_s[...] = jnp.zeros_like(w_s)

    q_row = q_ref[...]
    cm = m_s[...]
    cu = u_s[...]
    cw = w_s[...]

    C = _CHUNK
    ii = jax.lax.broadcasted_iota(jnp.int32, (C, C), 0)
    jj = jax.lax.broadcasted_iota(jnp.int32, (C, C), 1)
    tri = jj <= ii

    s_row = jax.lax.dot_general(
        q_row, k_ref[...], (((1,), (1,)), ((), ())),
        preferred_element_type=jnp.float32)

    for c in range(_BLK // _CHUNK):
        v_blk = v_ref[c * C:(c + 1) * C, :]
        S = jnp.broadcast_to(s_row[:, c * C:(c + 1) * C], (C, C))
        m_loc = jnp.max(jnp.where(tri, S, -jnp.inf), axis=1, keepdims=True)
        m_col = jnp.maximum(m_loc, cm)
        E = jnp.where(tri, jnp.exp(S - m_col), 0.0)
        ce = jnp.exp(cm - m_col)
        den = jnp.sum(E, axis=1, keepdims=True) + ce * cu
        num = jax.lax.dot_general(
            E, v_blk, (((1,), (0,)), ((), ())),
            preferred_element_type=jnp.float32)
        num = num + ce * cw
        o_ref[c * C:(c + 1) * C, :] = num / den
        cm = m_col[C - 1:C, :]
        cu = den[C - 1:C, :]
        cw = num[C - 1:C, :]

    m_s[...] = cm
    u_s[...] = cu
    w_s[...] = cw


def kernel(K, V, q):
    q2 = q.reshape(1, _D)
    grid = (_N // _BLK,)
    return pl.pallas_call(
        _scan_kernel,
        out_shape=jax.ShapeDtypeStruct((_N, _D), jnp.float32),
        grid=grid,
        in_specs=[
            pl.BlockSpec((1, _D), lambda i: (0, 0)),
            pl.BlockSpec((_BLK, _D), lambda i: (i, 0)),
            pl.BlockSpec((_BLK, _D), lambda i: (i, 0)),
        ],
        out_specs=pl.BlockSpec((_BLK, _D), lambda i: (i, 0)),
        scratch_shapes=[
            pltpu.VMEM((1, 1), jnp.float32),
            pltpu.VMEM((1, 1), jnp.float32),
            pltpu.VMEM((1, _D), jnp.float32),
        ],
        compiler_params=pltpu.CompilerParams(
            dimension_semantics=("arbitrary",),
            vmem_limit_bytes=52 * 1024 * 1024,
        ),
        name="aaren_scan",
    )(q2, K, V)


# speedup vs baseline: 1.3403x; 1.3403x over previous
import jax
import jax.numpy as jnp
from jax.experimental import pallas as pl
from jax.experimental.pallas import tpu as pltpu

_N = 131072
_D = 256
_BLK = 8192
_CHUNK = 512


def _scan_kernel(q_ref, k_ref, v_ref, o_ref, m_s, u_s, w_s):
    i = pl.program_id(0)

    @pl.when(i == 0)
    def _():
        m_s[...] = jnp.full_like(m_s, -jnp.inf)
        u_s[...] = jnp.zeros_like(u_s)
        w_s[...] = jnp.zeros_like(w_s)

    q_row = q_ref[...]
    cm = m_s[...]
    cu = u_s[...]
    cw = w_s[...]

    C = _CHUNK
    ii = jax.lax.broadcasted_iota(jnp.int32, (C, C), 0)
    jj = jax.lax.broadcasted_iota(jnp.int32, (C, C), 1)
    tri = jj <= ii

    s_row = jax.lax.dot_general(
        q_row, k_ref[...], (((1,), (1,)), ((), ())),
        preferred_element_type=jnp.float32)

    for c in range(_BLK // _CHUNK):
        v_blk = v_ref[c * C:(c + 1) * C, :]
        S = jnp.broadcast_to(s_row[:, c * C:(c + 1) * C], (C, C))
        m_loc = jnp.max(jnp.where(tri, S, -jnp.inf), axis=1, keepdims=True)
        m_col = jnp.maximum(m_loc, cm)
        E = jnp.where(tri, jnp.exp(S - m_col), 0.0)
        ce = jnp.exp(cm - m_col)
        den = jnp.sum(E, axis=1, keepdims=True) + ce * cu
        num = jax.lax.dot_general(
            E, v_blk, (((1,), (0,)), ((), ())),
            preferred_element_type=jnp.float32)
        num = num + ce * cw
        o_ref[c * C:(c + 1) * C, :] = num / den
        cm = m_col[C - 1:C, :]
        cu = den[C - 1:C, :]
        cw = num[C - 1:C, :]

    m_s[...] = cm
    u_s[...] = cu
    w_s[...] = cw


def kernel(K, V, q):
    q2 = q.reshape(1, _D)
    grid = (_N // _BLK,)
    return pl.pallas_call(
        _scan_kernel,
        out_shape=jax.ShapeDtypeStruct((_N, _D), jnp.float32),
        grid=grid,
        in_specs=[
            pl.BlockSpec((1, _D), lambda i: (0, 0)),
            pl.BlockSpec((_BLK, _D), lambda i: (i, 0)),
            pl.BlockSpec((_BLK, _D), lambda i: (i, 0)),
        ],
        out_specs=pl.BlockSpec((_BLK, _D), lambda i: (i, 0)),
        scratch_shapes=[
            pltpu.VMEM((1, 1), jnp.float32),
            pltpu.VMEM((1, 1), jnp.float32),
            pltpu.VMEM((1, _D), jnp.float32),
        ],
        compiler_params=pltpu.CompilerParams(
            dimension_semantics=("arbitrary",),
            vmem_limit_bytes=52 * 1024 * 1024,
        ),
        name="aaren_scan",
    )(q2, K, V)
